# baseline (speedup 1.0000x reference)
import functools

import jax
import jax.numpy as jnp
from jax import lax
from jax.experimental import pallas as pl
from jax.experimental.pallas import tpu as pltpu
from jax.experimental.pallas import tpu_sc as plsc

N_USER = 25000
N_ITEM = 25000
D = 64
DH = 32
E = 800000
B = 4096

NC = 2
NS = 16
NW = NC * NS

HALF = 25000
RP = 25088
NROW = 2 * RP

CH = 320
EP = 808960
EROWS = EP // CH
CPT = EROWS // NS
BLK = 2
NBLK = CPT // BLK

ZR = 224
STRIPE = NROW // NS


def _mesh():
    return plsc.VectorSubcoreMesh(core_axis_name="c", subcore_axis_name="s",
                                  num_cores=NC, num_subcores=NS)


def _one_layer(cid, sid, gsrc_hbm, ldst_hbm, w2_hbm, emb_in, emb_out,
               G, L, R, w_blk, acc_sh, GS, SS, TS, LS, wsem, csem):
    rows_a, rows_b = R
    gsrc_a, gsrc_b = G
    lidx_a, lidx_b = L
    gsem_a = GS[0]
    ssem_b = SS[1]
    stsem_b = TS[1]
    lsem_a = LS[0]

    def zb(i, _):
        for c2 in range(2):
            sl = pl.ds(c2 * 16, 16)
            rows_a[i, sl] = jnp.zeros((16,), jnp.float32)
            rows_b[i, sl] = jnp.zeros((16,), jnp.float32)
        return _
    lax.fori_loop(0, CH, zb, None)
    for k in range(STRIPE // ZR):
        pltpu.async_copy(rows_a.at[pl.ds(0, ZR)],
                         acc_sh.at[pl.ds(sid * STRIPE + k * ZR, ZR)], csem)
    for c8 in range(CH // 16):
        lidx_b[pl.ds(c8 * 16, 16)] = jnp.zeros((16,), jnp.int32)
    for k in range(STRIPE // ZR):
        pltpu.make_async_copy(
            rows_a.at[pl.ds(0, ZR)],
            acc_sh.at[pl.ds(sid * STRIPE + k * ZR, ZR)], csem).wait()
    plsc.subcore_barrier()

    c0 = sid * CPT

    def stage_gsrc(buf, sem, c):
        cc = jnp.minimum(c, EROWS - 1)
        return pltpu.async_copy(gsrc_hbm.at[pl.ds(cid * EP + cc * CH, CH)],
                                buf, sem)

    def stage_ldst(buf, sem, c):
        cc = jnp.minimum(c, EROWS - 1)
        return pltpu.async_copy(ldst_hbm.at[pl.ds(cc * CH, CH)], buf, sem)

    pltpu.sync_copy(gsrc_hbm.at[pl.ds(cid * EP + c0 * CH, CH)], gsrc_a)
    pltpu.async_copy(emb_in.at[gsrc_a], rows_a, gsem_a)
    stage_gsrc(gsrc_b, stsem_b, c0 + 1)
    stage_ldst(lidx_a, lsem_a, c0)
    pltpu.async_copy(rows_b, acc_sh.at[lidx_b], ssem_b, add=True)

    def stage_w(bk):
        bb = jnp.minimum(bk, NBLK - 1)
        woff = (bk % 2) * BLK
        return pltpu.async_copy(w2_hbm.at[pl.ds(c0 + bb * BLK, BLK)],
                                w_blk.at[pl.ds(woff, BLK)], wsem)
    stage_w(0)

    def block(bk, _):
        rb = c0 + bk * BLK
        woff = (bk % 2) * BLK
        pltpu.make_async_copy(w2_hbm.at[pl.ds(0, BLK)],
                              w_blk.at[pl.ds(0, BLK)], wsem).wait()
        stage_w(bk + 1)
        for j in range(BLK):
            p = j & 1
            q = 1 - p
            c = rb + j
            pltpu.make_async_copy(emb_in.at[G[p]], R[p], GS[p]).wait()
            pltpu.make_async_copy(R[q], acc_sh.at[L[q]], SS[q]).wait()
            pltpu.make_async_copy(gsrc_hbm.at[pl.ds(0, CH)], G[q],
                                  TS[q]).wait()
            pltpu.async_copy(emb_in.at[G[q]], R[q], GS[q])
            stage_gsrc(G[p], TS[p], c + 2)
            stage_ldst(L[q], LS[q], c + 1)
            def scale(g, _, j=j, p=p):
                wv = w_blk[woff + j, pl.ds(g * 16, 16)]
                e_ = g * 16
                for i in range(16):
                    ws = wv[i]
                    for qd in range(2):
                        sl = pl.ds(qd * 16, 16)
                        R[p][e_ + i, sl] = R[p][e_ + i, sl] * ws
                return _
            lax.fori_loop(0, CH // 16, scale, None)
            pltpu.make_async_copy(ldst_hbm.at[pl.ds(0, CH)], L[p],
                                  LS[p]).wait()
            pltpu.async_copy(R[p], acc_sh.at[L[p]], SS[p], add=True)
        return _
    lax.fori_loop(0, NBLK, block, None)

    pltpu.make_async_copy(emb_in.at[gsrc_a], rows_a, gsem_a).wait()
    pltpu.make_async_copy(gsrc_hbm.at[pl.ds(0, CH)], gsrc_b, stsem_b).wait()
    pltpu.make_async_copy(ldst_hbm.at[pl.ds(0, CH)], lidx_a, lsem_a).wait()
    pltpu.make_async_copy(rows_b, acc_sh.at[lidx_b], ssem_b).wait()
    pltpu.make_async_copy(w2_hbm.at[pl.ds(0, BLK)],
                          w_blk.at[pl.ds(0, BLK)], wsem).wait()

    plsc.subcore_barrier()
    out_base = cid * NROW + sid * STRIPE
    pltpu.sync_copy(acc_sh.at[pl.ds(sid * STRIPE, STRIPE)],
                    emb_out.at[pl.ds(out_base, STRIPE)])
    plsc.subcore_barrier()


def _layers_body(gsrc_hbm, ldst_hbm, w2_hbm, emb0, emb1, emb2, emb3,
                 gsrc_a, gsrc_b, lidx_a, lidx_b, rows_a, rows_b,
                 w_blk, acc_sh,
                 gsem_a, gsem_b, ssem_a, ssem_b, stsem_a, stsem_b,
                 lsem_a, lsem_b, wsem, csem):
    cid = lax.axis_index("c")
    sid = lax.axis_index("s")
    G = (gsrc_a, gsrc_b)
    L = (lidx_a, lidx_b)
    R = (rows_a, rows_b)
    GS = (gsem_a, gsem_b)
    SS = (ssem_a, ssem_b)
    TS = (stsem_a, stsem_b)
    LS = (lsem_a, lsem_b)
    for ein, eout in ((emb0, emb1), (emb1, emb2), (emb2, emb3)):
        _one_layer(cid, sid, gsrc_hbm, ldst_hbm, w2_hbm, ein, eout,
                   G, L, R, w_blk, acc_sh, GS, SS, TS, LS, wsem, csem)


@functools.partial(jax.jit, static_argnames=())
def _layers(gsrc_cat, ldst1, w2, emb_in):
    return pl.kernel(
        _layers_body,
        out_type=[jax.ShapeDtypeStruct((2 * NROW, DH), jnp.float32)] * 3,
        mesh=_mesh(),
        compiler_params=pltpu.CompilerParams(use_tc_tiling_on_sc=False),
        scratch_types=[
            pltpu.VMEM((CH,), jnp.int32),
            pltpu.VMEM((CH,), jnp.int32),
            pltpu.VMEM((CH,), jnp.int32),
            pltpu.VMEM((CH,), jnp.int32),
            pltpu.VMEM((CH, DH), jnp.float32),
            pltpu.VMEM((CH, DH), jnp.float32),
            pltpu.VMEM((2 * BLK, CH), jnp.float32),
            pltpu.VMEM_SHARED((NROW, DH), jnp.float32),
            pltpu.SemaphoreType.DMA,
            pltpu.SemaphoreType.DMA,
            pltpu.SemaphoreType.DMA,
            pltpu.SemaphoreType.DMA,
            pltpu.SemaphoreType.DMA,
            pltpu.SemaphoreType.DMA,
            pltpu.SemaphoreType.DMA,
            pltpu.SemaphoreType.DMA,
            pltpu.SemaphoreType.DMA,
            pltpu.SemaphoreType.DMA,
        ],
    )(gsrc_cat, ldst1, w2, emb_in)


QT = B // NW


def _score_body(user_hbm, item_hbm, e0, e1, e2, e3, score_hbm,
                uidx_v, uidx2_v, iidx_v, iidx2_v,
                uaA, uaB, iaA, iaB, ubA, ubB, ibA, ibB, sc_v, gsem):
    cid = lax.axis_index("c")
    sid = lax.axis_index("s")
    qbase = (sid * NC + cid) * QT
    pltpu.sync_copy(user_hbm.at[pl.ds(qbase, QT)], uidx_v)
    pltpu.sync_copy(item_hbm.at[pl.ds(qbase, QT)], iidx_v)
    for c in range(QT // 16):
        sl = pl.ds(c * 16, 16)
        iidx_v[sl] = iidx_v[sl] + RP
        uidx2_v[sl] = uidx_v[sl] + NROW
        iidx2_v[sl] = iidx_v[sl] + NROW
    d0 = pltpu.async_copy(e0.at[uidx_v], uaA, gsem)
    d1 = pltpu.async_copy(e0.at[uidx2_v], uaB, gsem)
    d2 = pltpu.async_copy(e0.at[iidx_v], iaA, gsem)
    d3 = pltpu.async_copy(e0.at[iidx2_v], iaB, gsem)
    d0.wait(); d1.wait(); d2.wait(); d3.wait()
    for emb in (e1, e2, e3):
        d0 = pltpu.async_copy(emb.at[uidx_v], ubA, gsem)
        d1 = pltpu.async_copy(emb.at[uidx2_v], ubB, gsem)
        d2 = pltpu.async_copy(emb.at[iidx_v], ibA, gsem)
        d3 = pltpu.async_copy(emb.at[iidx2_v], ibB, gsem)
        d0.wait(); d1.wait(); d2.wait(); d3.wait()
        def accum(e, _):
            for q in range(2):
                sl = pl.ds(q * 16, 16)
                uaA[e, sl] = uaA[e, sl] + ubA[e, sl]
                uaB[e, sl] = uaB[e, sl] + ubB[e, sl]
                iaA[e, sl] = iaA[e, sl] + ibA[e, sl]
                iaB[e, sl] = iaB[e, sl] + ibB[e, sl]
            return _
        lax.fori_loop(0, QT, accum, None)

    lane = lax.iota(jnp.int32, 16)
    def dot(g, _):
        vec = jnp.zeros((16,), jnp.float32)
        for i in range(16):
            e = g * 16 + i
            acc = jnp.zeros((16,), jnp.float32)
            for q in range(2):
                sl = pl.ds(q * 16, 16)
                acc = acc + uaA[e, sl] * iaA[e, sl]
                acc = acc + uaB[e, sl] * iaB[e, sl]
            s_ = acc[0]
            for k in range(1, 16):
                s_ = s_ + acc[k]
            vec = jnp.where(lane == i, s_, vec)
        sc_v[pl.ds(g * 16, 16)] = vec * jnp.float32(1.0 / 16.0)
        return _
    lax.fori_loop(0, QT // 16, dot, None)
    pltpu.sync_copy(sc_v, score_hbm.at[pl.ds(qbase, QT)])


def _score(user, item, e0, e1, e2, e3):
    return pl.kernel(
        _score_body,
        out_type=jax.ShapeDtypeStruct((B,), jnp.float32),
        mesh=_mesh(),
        compiler_params=pltpu.CompilerParams(use_tc_tiling_on_sc=False),
        scratch_types=[
            pltpu.VMEM((QT,), jnp.int32),
            pltpu.VMEM((QT,), jnp.int32),
            pltpu.VMEM((QT,), jnp.int32),
            pltpu.VMEM((QT,), jnp.int32),
            pltpu.VMEM((QT, DH), jnp.float32),
            pltpu.VMEM((QT, DH), jnp.float32),
            pltpu.VMEM((QT, DH), jnp.float32),
            pltpu.VMEM((QT, DH), jnp.float32),
            pltpu.VMEM((QT, DH), jnp.float32),
            pltpu.VMEM((QT, DH), jnp.float32),
            pltpu.VMEM((QT, DH), jnp.float32),
            pltpu.VMEM((QT, DH), jnp.float32),
            pltpu.VMEM((QT,), jnp.float32),
            pltpu.SemaphoreType.DMA,
        ],
    )(user, item, e0, e1, e2, e3)


def kernel(user, item, edge_index, edge_weight, user_emb, item_emb):
    src = edge_index[0].astype(jnp.int32)
    dst = edge_index[1].astype(jnp.int32)
    gsrc = src + jnp.where(src >= HALF, jnp.int32(RP - HALF), jnp.int32(0))
    ldst = dst + jnp.where(dst >= HALF, jnp.int32(RP - HALF), jnp.int32(0))
    pad_i = jnp.zeros((EP - E,), jnp.int32)
    pad_f = jnp.zeros((EP - E,), jnp.float32)
    gsrc_p = jnp.concatenate([gsrc, pad_i])
    gsrc_cat = jnp.concatenate([gsrc_p, gsrc_p + jnp.int32(NROW)])
    ldst1 = jnp.concatenate([ldst, pad_i])
    w2 = jnp.concatenate([edge_weight.astype(jnp.float32), pad_f]).reshape(EROWS, CH)

    padrows = jnp.zeros((RP - HALF, DH), jnp.float32)
    emb0 = jnp.concatenate(
        [user_emb[:, :DH], padrows, item_emb[:, :DH], padrows,
         user_emb[:, DH:], padrows, item_emb[:, DH:], padrows], axis=0)

    emb1, emb2, emb3 = _layers(gsrc_cat, ldst1, w2, emb0)
    return _score(user.astype(jnp.int32), item.astype(jnp.int32),
                  emb0, emb1, emb2, emb3)


# speedup vs baseline: 1.1909x; 1.1909x over previous
import functools

import jax
import jax.numpy as jnp
from jax import lax
from jax.experimental import pallas as pl
from jax.experimental.pallas import tpu as pltpu
from jax.experimental.pallas import tpu_sc as plsc

N_USER = 25000
N_ITEM = 25000
D = 64
DH = 32
E = 800000
B = 4096

NC = 2
NS = 16
NW = NC * NS

HALF = 25000
RP = 25088
NROW = 2 * RP

CH = 256
EP = 802816
EROWS = EP // CH
CPT = EROWS // NS
BLK = 4
NBLK = CPT // BLK

ZR = 224
STRIPE = NROW // NS


def _mesh():
    return plsc.VectorSubcoreMesh(core_axis_name="c", subcore_axis_name="s",
                                  num_cores=NC, num_subcores=NS)


def _one_layer(cid, sid, gsrc_hbm, ldst_hbm, w2_hbm, emb_in, emb_out,
               G, L, R, w_blk, acc_sh, GS, SS, TS, LS, wsem, csem):
    rows_a, rows_b = R
    gsrc_a, gsrc_b = G
    lidx_a, lidx_b = L
    gsem_a = GS[0]
    ssem_b = SS[1]
    stsem_b = TS[1]
    lsem_a = LS[0]

    def zb(i, _):
        for c2 in range(2):
            sl = pl.ds(c2 * 16, 16)
            rows_a[i, sl] = jnp.zeros((16,), jnp.float32)
            rows_b[i, sl] = jnp.zeros((16,), jnp.float32)
        return _
    lax.fori_loop(0, CH, zb, None)
    for k in range(STRIPE // ZR):
        pltpu.async_copy(rows_a.at[pl.ds(0, ZR)],
                         acc_sh.at[pl.ds(sid * STRIPE + k * ZR, ZR)], csem)
    for c8 in range(CH // 16):
        lidx_b[pl.ds(c8 * 16, 16)] = jnp.zeros((16,), jnp.int32)
    for k in range(STRIPE // ZR):
        pltpu.make_async_copy(
            rows_a.at[pl.ds(0, ZR)],
            acc_sh.at[pl.ds(sid * STRIPE + k * ZR, ZR)], csem).wait()
    plsc.subcore_barrier()

    c0 = sid * CPT

    def stage_gsrc(buf, sem, c):
        cc = jnp.minimum(c, EROWS - 1)
        return pltpu.async_copy(gsrc_hbm.at[pl.ds(cid * EP + cc * CH, CH)],
                                buf, sem)

    def stage_ldst(buf, sem, c):
        cc = jnp.minimum(c, EROWS - 1)
        return pltpu.async_copy(ldst_hbm.at[pl.ds(cc * CH, CH)], buf, sem)

    pltpu.sync_copy(gsrc_hbm.at[pl.ds(cid * EP + c0 * CH, CH)], gsrc_a)
    pltpu.async_copy(emb_in.at[gsrc_a], rows_a, gsem_a)
    stage_gsrc(gsrc_b, stsem_b, c0 + 1)
    stage_ldst(lidx_a, lsem_a, c0)
    pltpu.async_copy(rows_b, acc_sh.at[lidx_b], ssem_b, add=True)

    def stage_w(bk):
        bb = jnp.minimum(bk, NBLK - 1)
        woff = (bk % 2) * BLK
        return pltpu.async_copy(w2_hbm.at[pl.ds(c0 + bb * BLK, BLK)],
                                w_blk.at[pl.ds(woff, BLK)], wsem)
    stage_w(0)

    def block(bk, _):
        rb = c0 + bk * BLK
        woff = (bk % 2) * BLK
        pltpu.make_async_copy(w2_hbm.at[pl.ds(0, BLK)],
                              w_blk.at[pl.ds(0, BLK)], wsem).wait()
        stage_w(bk + 1)
        for j in range(BLK):
            p = j & 1
            q = 1 - p
            c = rb + j
            pltpu.make_async_copy(emb_in.at[G[p]], R[p], GS[p]).wait()
            pltpu.make_async_copy(R[q], acc_sh.at[L[q]], SS[q]).wait()
            pltpu.make_async_copy(gsrc_hbm.at[pl.ds(0, CH)], G[q],
                                  TS[q]).wait()
            pltpu.async_copy(emb_in.at[G[q]], R[q], GS[q])
            stage_gsrc(G[p], TS[p], c + 2)
            stage_ldst(L[q], LS[q], c + 1)
            def scale(g, _, j=j, p=p):
                wv = w_blk[woff + j, pl.ds(g * 16, 16)]
                e_ = g * 16
                for i in range(16):
                    ws = wv[i]
                    for qd in range(2):
                        sl = pl.ds(qd * 16, 16)
                        R[p][e_ + i, sl] = R[p][e_ + i, sl] * ws
                return _
            lax.fori_loop(0, CH // 16, scale, None)
            pltpu.make_async_copy(ldst_hbm.at[pl.ds(0, CH)], L[p],
                                  LS[p]).wait()
            pltpu.async_copy(R[p], acc_sh.at[L[p]], SS[p], add=True)
        return _
    lax.fori_loop(0, NBLK, block, None)

    pltpu.make_async_copy(emb_in.at[gsrc_a], rows_a, gsem_a).wait()
    pltpu.make_async_copy(gsrc_hbm.at[pl.ds(0, CH)], gsrc_b, stsem_b).wait()
    pltpu.make_async_copy(ldst_hbm.at[pl.ds(0, CH)], lidx_a, lsem_a).wait()
    pltpu.make_async_copy(rows_b, acc_sh.at[lidx_b], ssem_b).wait()
    pltpu.make_async_copy(w2_hbm.at[pl.ds(0, BLK)],
                          w_blk.at[pl.ds(0, BLK)], wsem).wait()

    plsc.subcore_barrier()
    out_base = cid * NROW + sid * STRIPE
    pltpu.sync_copy(acc_sh.at[pl.ds(sid * STRIPE, STRIPE)],
                    emb_out.at[pl.ds(out_base, STRIPE)])
    plsc.subcore_barrier()


def _layers_body(gsrc_hbm, ldst_hbm, w2_hbm, emb0, emb1, emb2, emb3,
                 gsrc_a, gsrc_b, lidx_a, lidx_b, rows_a, rows_b,
                 w_blk, acc_sh,
                 gsem_a, gsem_b, ssem_a, ssem_b, stsem_a, stsem_b,
                 lsem_a, lsem_b, wsem, csem):
    cid = lax.axis_index("c")
    sid = lax.axis_index("s")
    G = (gsrc_a, gsrc_b)
    L = (lidx_a, lidx_b)
    R = (rows_a, rows_b)
    GS = (gsem_a, gsem_b)
    SS = (ssem_a, ssem_b)
    TS = (stsem_a, stsem_b)
    LS = (lsem_a, lsem_b)
    for ein, eout in ((emb0, emb1), (emb1, emb2), (emb2, emb3)):
        _one_layer(cid, sid, gsrc_hbm, ldst_hbm, w2_hbm, ein, eout,
                   G, L, R, w_blk, acc_sh, GS, SS, TS, LS, wsem, csem)


@functools.partial(jax.jit, static_argnames=())
def _layers(gsrc_cat, ldst1, w2, emb_in):
    return pl.kernel(
        _layers_body,
        out_type=[jax.ShapeDtypeStruct((2 * NROW, DH), jnp.float32)] * 3,
        mesh=_mesh(),
        compiler_params=pltpu.CompilerParams(use_tc_tiling_on_sc=False),
        scratch_types=[
            pltpu.VMEM((CH,), jnp.int32),
            pltpu.VMEM((CH,), jnp.int32),
            pltpu.VMEM((CH,), jnp.int32),
            pltpu.VMEM((CH,), jnp.int32),
            pltpu.VMEM((CH, DH), jnp.float32),
            pltpu.VMEM((CH, DH), jnp.float32),
            pltpu.VMEM((2 * BLK, CH), jnp.float32),
            pltpu.VMEM_SHARED((NROW, DH), jnp.float32),
            pltpu.SemaphoreType.DMA,
            pltpu.SemaphoreType.DMA,
            pltpu.SemaphoreType.DMA,
            pltpu.SemaphoreType.DMA,
            pltpu.SemaphoreType.DMA,
            pltpu.SemaphoreType.DMA,
            pltpu.SemaphoreType.DMA,
            pltpu.SemaphoreType.DMA,
            pltpu.SemaphoreType.DMA,
            pltpu.SemaphoreType.DMA,
        ],
    )(gsrc_cat, ldst1, w2, emb_in)


QT = B // NW


def _score_body(user_hbm, item_hbm, e0, e1, e2, e3, score_hbm,
                uidx_v, uidx2_v, iidx_v, iidx2_v,
                uaA, uaB, iaA, iaB, ubA, ubB, ibA, ibB, sc_v, gsem):
    cid = lax.axis_index("c")
    sid = lax.axis_index("s")
    qbase = (sid * NC + cid) * QT
    pltpu.sync_copy(user_hbm.at[pl.ds(qbase, QT)], uidx_v)
    pltpu.sync_copy(item_hbm.at[pl.ds(qbase, QT)], iidx_v)
    for c in range(QT // 16):
        sl = pl.ds(c * 16, 16)
        iidx_v[sl] = iidx_v[sl] + RP
        uidx2_v[sl] = uidx_v[sl] + NROW
        iidx2_v[sl] = iidx_v[sl] + NROW
    d0 = pltpu.async_copy(e0.at[uidx_v], uaA, gsem)
    d1 = pltpu.async_copy(e0.at[uidx2_v], uaB, gsem)
    d2 = pltpu.async_copy(e0.at[iidx_v], iaA, gsem)
    d3 = pltpu.async_copy(e0.at[iidx2_v], iaB, gsem)
    d0.wait(); d1.wait(); d2.wait(); d3.wait()
    for emb in (e1, e2, e3):
        d0 = pltpu.async_copy(emb.at[uidx_v], ubA, gsem)
        d1 = pltpu.async_copy(emb.at[uidx2_v], ubB, gsem)
        d2 = pltpu.async_copy(emb.at[iidx_v], ibA, gsem)
        d3 = pltpu.async_copy(emb.at[iidx2_v], ibB, gsem)
        d0.wait(); d1.wait(); d2.wait(); d3.wait()
        def accum(e, _):
            for q in range(2):
                sl = pl.ds(q * 16, 16)
                uaA[e, sl] = uaA[e, sl] + ubA[e, sl]
                uaB[e, sl] = uaB[e, sl] + ubB[e, sl]
                iaA[e, sl] = iaA[e, sl] + ibA[e, sl]
                iaB[e, sl] = iaB[e, sl] + ibB[e, sl]
            return _
        lax.fori_loop(0, QT, accum, None)

    lane = lax.iota(jnp.int32, 16)
    def dot(g, _):
        vec = jnp.zeros((16,), jnp.float32)
        for i in range(16):
            e = g * 16 + i
            acc = jnp.zeros((16,), jnp.float32)
            for q in range(2):
                sl = pl.ds(q * 16, 16)
                acc = acc + uaA[e, sl] * iaA[e, sl]
                acc = acc + uaB[e, sl] * iaB[e, sl]
            s_ = acc[0]
            for k in range(1, 16):
                s_ = s_ + acc[k]
            vec = jnp.where(lane == i, s_, vec)
        sc_v[pl.ds(g * 16, 16)] = vec * jnp.float32(1.0 / 16.0)
        return _
    lax.fori_loop(0, QT // 16, dot, None)
    pltpu.sync_copy(sc_v, score_hbm.at[pl.ds(qbase, QT)])


def _score(user, item, e0, e1, e2, e3):
    return pl.kernel(
        _score_body,
        out_type=jax.ShapeDtypeStruct((B,), jnp.float32),
        mesh=_mesh(),
        compiler_params=pltpu.CompilerParams(use_tc_tiling_on_sc=False),
        scratch_types=[
            pltpu.VMEM((QT,), jnp.int32),
            pltpu.VMEM((QT,), jnp.int32),
            pltpu.VMEM((QT,), jnp.int32),
            pltpu.VMEM((QT,), jnp.int32),
            pltpu.VMEM((QT, DH), jnp.float32),
            pltpu.VMEM((QT, DH), jnp.float32),
            pltpu.VMEM((QT, DH), jnp.float32),
            pltpu.VMEM((QT, DH), jnp.float32),
            pltpu.VMEM((QT, DH), jnp.float32),
            pltpu.VMEM((QT, DH), jnp.float32),
            pltpu.VMEM((QT, DH), jnp.float32),
            pltpu.VMEM((QT, DH), jnp.float32),
            pltpu.VMEM((QT,), jnp.float32),
            pltpu.SemaphoreType.DMA,
        ],
    )(user, item, e0, e1, e2, e3)


def kernel(user, item, edge_index, edge_weight, user_emb, item_emb):
    src = edge_index[0].astype(jnp.int32)
    dst = edge_index[1].astype(jnp.int32)
    gsrc = src + jnp.where(src >= HALF, jnp.int32(RP - HALF), jnp.int32(0))
    ldst = dst + jnp.where(dst >= HALF, jnp.int32(RP - HALF), jnp.int32(0))
    pad_i = jnp.zeros((EP - E,), jnp.int32)
    pad_f = jnp.zeros((EP - E,), jnp.float32)
    gsrc_p = jnp.concatenate([gsrc, pad_i])
    gsrc_cat = jnp.concatenate([gsrc_p, gsrc_p + jnp.int32(NROW)])
    ldst1 = jnp.concatenate([ldst, pad_i])
    w2 = jnp.concatenate([edge_weight.astype(jnp.float32), pad_f]).reshape(EROWS, CH)

    padrows = jnp.zeros((RP - HALF, DH), jnp.float32)
    emb0 = jnp.concatenate(
        [user_emb[:, :DH], padrows, item_emb[:, :DH], padrows,
         user_emb[:, DH:], padrows, item_emb[:, DH:], padrows], axis=0)

    emb1, emb2, emb3 = _layers(gsrc_cat, ldst1, w2, emb0)
    return _score(user.astype(jnp.int32), item.astype(jnp.int32),
                  emb0, emb1, emb2, emb3)
